# baseline (speedup 1.0000x reference)
import jax
import jax.numpy as jnp
from jax import lax
from jax.experimental import pallas as pl
from jax.experimental.pallas import tpu as pltpu
from jax.experimental.pallas import tpu_sc as plsc

VOCAB = 1000000
EMBED = 32
SEQ = 200
BATCH = 4096

NC = 2
NS = 16
NW = NC * NS

BBLK = BATCH // NW
RING = 4
AHEAD = 2

TW = 2048


def _tc_table_transpose(wt_t):
    grid = pl.cdiv(VOCAB, TW)

    def body(x_ref, o_ref):
        y = x_ref[...].reshape(EMBED, TW // 4, 4)
        for j in range(4):
            o_ref[:, EMBED * j:EMBED * (j + 1)] = y[:, :, j].T

    return pl.pallas_call(
        body,
        grid=(grid,),
        in_specs=[pl.BlockSpec((EMBED, TW), lambda g: (0, g))],
        out_specs=pl.BlockSpec((TW // 4, 128), lambda g: (g, 0)),
        out_shape=jax.ShapeDtypeStruct((VOCAB // 4, 128), jnp.float32),
    )(wt_t)


def _sc_embed(tok_t, word_lin, pos_t):
    mesh = plsc.VectorSubcoreMesh(core_axis_name="c", subcore_axis_name="s",
                                  num_cores=NC, num_subcores=NS)

    def body(tok_hbm, word_hbm, pos_hbm, out_hbm, idx_v, pos_v,
             *bufs_and_sems):
        gbufs = bufs_and_sems[:RING]
        obufs = bufs_and_sems[RING:2 * RING]
        gsem = bufs_and_sems[2 * RING:3 * RING]
        osem = bufs_and_sems[3 * RING:4 * RING]
        wid = lax.axis_index("s") * NC + lax.axis_index("c")
        pltpu.sync_copy(tok_hbm.at[:, pl.ds(wid * BBLK, BBLK)], idx_v)
        pltpu.sync_copy(pos_hbm.at[:, pl.ds(0, SEQ)], pos_v)

        riv = [lax.iota(jnp.int32, 16) + 16 * g for g in range(BBLK // 16)]

        def gather(k, slot):
            return pltpu.make_async_copy(word_hbm.at[idx_v.at[k]],
                                         gbufs[slot], gsem[slot])

        def wback(k, slot):
            dst = out_hbm.at[k, :, pl.ds(wid, 1)]
            return pltpu.make_async_copy(obufs[slot], dst, osem[slot])

        for k in range(AHEAD):
            gather(k, k).start()

        def outer(k0, carry):
            for b in range(RING):
                k = k0 * RING + b
                slot_w = (b + AHEAD) % RING
                if b >= RING - AHEAD:
                    wback(k - (RING - AHEAD), slot_w).wait()
                else:
                    @pl.when(k0 > 0)
                    def _():
                        wback(k - (RING - AHEAD), slot_w).wait()

                @pl.when(k + AHEAD < SEQ)
                def _():
                    gather(k + AHEAD, slot_w).start()

                gather(k, b).wait()
                gbuf, obuf = gbufs[b], obufs[b]
                ks = jnp.full((16,), k, jnp.int32)
                for dt in range(EMBED // 8):
                    for di in range(8):
                        d = dt * 8 + di
                        dsp = jnp.full((16,), d, jnp.int32)
                        pv = plsc.load_gather(pos_v, [dsp, ks])
                        for g in range(BBLK // 16):
                            vals = plsc.load_gather(gbuf, [riv[g], dsp])
                            obuf[dt, 0, di, pl.ds(16 * g, 16)] = vals + pv
                wback(k, b).start()
            return carry

        lax.fori_loop(0, SEQ // RING, outer, 0)
        for k in range(SEQ - (RING - AHEAD), SEQ):
            wback(k, k % RING).wait()

    f = pl.kernel(
        body,
        out_type=jax.ShapeDtypeStruct((SEQ, EMBED // 8, NW, 8, BBLK),
                                      jnp.float32),
        mesh=mesh,
        scratch_types=(
            [pltpu.VMEM((SEQ, BBLK), jnp.int32),
             pltpu.VMEM((EMBED, SEQ), jnp.float32)]
            + [pltpu.VMEM((BBLK, EMBED), jnp.float32) for _ in range(RING)]
            + [pltpu.VMEM((EMBED // 8, 1, 8, BBLK), jnp.float32)
               for _ in range(RING)]
            + [pltpu.SemaphoreType.DMA for _ in range(2 * RING)]
        ),
        compiler_params=pltpu.CompilerParams(use_tc_tiling_on_sc=False,
                                             needs_layout_passes=False),
    )
    return f(tok_t, word_lin, pos_t)


def kernel(token_ids, word_table, pos_table):
    tok_t = token_ids.astype(jnp.int32).T
    word_lin = _tc_table_transpose(word_table.T).reshape(VOCAB, EMBED)
    pos_t = pos_table.T
    out5 = _sc_embed(tok_t, word_lin, pos_t)
    return out5.transpose(2, 4, 0, 1, 3).reshape(BATCH, SEQ, EMBED)


# speedup vs baseline: 7.8079x; 7.8079x over previous
import jax
import jax.numpy as jnp
from jax import lax
from jax.experimental import pallas as pl
from jax.experimental.pallas import tpu as pltpu
from jax.experimental.pallas import tpu_sc as plsc

VOCAB = 1000000
EMBED = 32
SEQ = 200
BATCH = 4096

NC = 2
NS = 16
NW = NC * NS

BBLK = BATCH // NW
RING = 4
AHEAD = 2

OPAD = BBLK + 1


def _sc_embed(tok_t, word_lin, pos_t):
    mesh = plsc.VectorSubcoreMesh(core_axis_name="c", subcore_axis_name="s",
                                  num_cores=NC, num_subcores=NS)

    def body(tok_hbm, word_hbm, pos_hbm, out_hbm, idx_v, pos_v,
             *bufs_and_sems):
        gbufs = bufs_and_sems[:RING]
        obufs = bufs_and_sems[RING:2 * RING]
        gsem = bufs_and_sems[2 * RING:3 * RING]
        osem = bufs_and_sems[3 * RING:4 * RING]
        wid = lax.axis_index("s") * NC + lax.axis_index("c")
        pltpu.sync_copy(tok_hbm.at[:, pl.ds(wid * BBLK, BBLK)], idx_v)
        pltpu.sync_copy(pos_hbm.at[pl.ds(0, SEQ)], pos_v)

        dvec = [lax.iota(jnp.int32, 16) + 16 * h for h in range(EMBED // 16)]
        dtv = [d // 8 for d in dvec]
        div = [lax.rem(d, 8) for d in dvec]
        zv = jnp.zeros((16,), jnp.int32)

        def gather(k, slot):
            return pltpu.make_async_copy(word_hbm.at[idx_v.at[k]],
                                         gbufs[slot], gsem[slot])

        def wback(k, slot):
            src = obufs[slot].at[:, :, :, pl.ds(0, BBLK)]
            dst = out_hbm.at[k, :, pl.ds(wid, 1)]
            return pltpu.make_async_copy(src, dst, osem[slot])

        for k in range(AHEAD):
            gather(k, k).start()

        def outer(k0, carry):
            for b in range(RING):
                k = k0 * RING + b
                slot_w = (b + AHEAD) % RING
                if b >= RING - AHEAD:
                    wback(k - (RING - AHEAD), slot_w).wait()
                else:
                    @pl.when(k0 > 0)
                    def _():
                        wback(k - (RING - AHEAD), slot_w).wait()

                @pl.when(k + AHEAD < SEQ)
                def _():
                    gather(k + AHEAD, slot_w).start()

                gather(k, b).wait()
                gbuf, obuf = gbufs[b], obufs[b]
                pv = [pos_v[k, pl.ds(16 * h, 16)]
                      for h in range(EMBED // 16)]
                for r in range(BBLK):
                    rv = jnp.full((16,), r, jnp.int32)
                    for h in range(EMBED // 16):
                        vals = gbuf[r, pl.ds(16 * h, 16)] + pv[h]
                        plsc.store_scatter(obuf, [dtv[h], zv, div[h], rv],
                                           vals)
                wback(k, b).start()
            return carry

        lax.fori_loop(0, SEQ // RING, outer, 0)
        for k in range(SEQ - (RING - AHEAD), SEQ):
            wback(k, k % RING).wait()

    f = pl.kernel(
        body,
        out_type=jax.ShapeDtypeStruct((SEQ, EMBED // 8, NW, 8, BBLK),
                                      jnp.float32),
        mesh=mesh,
        scratch_types=(
            [pltpu.VMEM((SEQ, BBLK), jnp.int32),
             pltpu.VMEM((SEQ, EMBED), jnp.float32)]
            + [pltpu.VMEM((BBLK, EMBED), jnp.float32) for _ in range(RING)]
            + [pltpu.VMEM((EMBED // 8, 1, 8, OPAD), jnp.float32)
               for _ in range(RING)]
            + [pltpu.SemaphoreType.DMA for _ in range(2 * RING)]
        ),
        compiler_params=pltpu.CompilerParams(use_tc_tiling_on_sc=False,
                                             needs_layout_passes=False),
    )
    return f(tok_t, word_lin, pos_t)


def kernel(token_ids, word_table, pos_table):
    tok_t = token_ids.astype(jnp.int32).T * 4
    word_lin = jnp.pad(word_table, ((0, 0), (0, 128 - EMBED))
                       ).reshape(4 * VOCAB, EMBED)
    out5 = _sc_embed(tok_t, word_lin, pos_table)
    return out5.transpose(2, 4, 0, 1, 3).reshape(BATCH, SEQ, EMBED)
